# baseline (speedup 1.0000x reference)
import jax
import jax.numpy as jnp
from jax import lax
from jax.experimental import pallas as pl
from jax.experimental.pallas import tpu as pltpu
from jax.experimental.pallas import tpu_sc as plsc

NC = 2
NS = 16
L = 16
NW = NC * NS

B = 16384
D = 64
D2 = 2 * D
H = 256
BPW = B // NW

CH = 256
NCHUNK = BPW // CH

BLK = 2048
GRID = B // BLK


def _mix_body(uidx_hbm, iidx_hbm, utab_hbm, itab_hbm, out_hbm,
              uidx_c, iidx_c, half_c, upair_v, ipair_v, mix_v, usem, isem):
    wid = lax.axis_index("s") * NC + lax.axis_index("c")
    base = wid * BPW

    def chunk(c, carry0):
        cbase = base + c * CH
        pltpu.sync_copy(uidx_hbm.at[pl.ds(cbase, CH)], uidx_c)
        pltpu.sync_copy(iidx_hbm.at[pl.ds(cbase, CH)], iidx_c)

        def halve(k, carry):
            sl = pl.ds(k * L, L)
            u = uidx_c[sl]
            i = iidx_c[sl]
            half_c[sl] = (u & 1) | ((i & 1) << 1)
            uidx_c[sl] = u >> 1
            iidx_c[sl] = i >> 1
            return carry

        lax.fori_loop(0, CH // L, halve, 0)

        cu = pltpu.async_copy(utab_hbm.at[uidx_c], upair_v, usem)
        ci = pltpu.async_copy(itab_hbm.at[iidx_c], ipair_v, isem)
        cu.wait()
        ci.wait()

        def group(k, carry):
            hvec = half_c[pl.ds(k * L, L)]
            for j in range(L):
                h = hvec[j]
                uh = h & 1
                ih = h >> 1
                row = k * L + j
                for t in range(D // L):
                    ulo = upair_v[row, pl.ds(t * L, L)]
                    uhi = upair_v[row, pl.ds(D + t * L, L)]
                    ilo = ipair_v[row, pl.ds(t * L, L)]
                    ihi = ipair_v[row, pl.ds(D + t * L, L)]
                    uval = jnp.where(uh == 1, uhi, ulo)
                    ival = jnp.where(ih == 1, ihi, ilo)
                    mix_v[row, pl.ds(t * L, L)] = uval * ival
            return carry

        lax.fori_loop(0, CH // L, group, 0)

        def row_out(i, carry):
            pltpu.async_copy(mix_v.at[pl.ds(i, 1)],
                             out_hbm.at[pl.ds(cbase + i, 1)], usem)
            return carry

        lax.fori_loop(0, CH, row_out, 0)
        pltpu.make_async_copy(out_hbm.at[pl.ds(0, CH)], mix_v, usem).wait()
        return carry0

    lax.fori_loop(0, NCHUNK, chunk, 0)


_mix_call = pl.kernel(
    _mix_body,
    mesh=plsc.VectorSubcoreMesh(core_axis_name="c", subcore_axis_name="s"),
    out_type=jax.ShapeDtypeStruct((B, D), jnp.float32),
    scratch_types=[
        pltpu.VMEM((CH,), jnp.int32),
        pltpu.VMEM((CH,), jnp.int32),
        pltpu.VMEM((CH,), jnp.int32),
        pltpu.VMEM((CH, D2), jnp.float32),
        pltpu.VMEM((CH, D2), jnp.float32),
        pltpu.VMEM((CH, D), jnp.float32),
        pltpu.SemaphoreType.DMA,
        pltpu.SemaphoreType.DMA,
    ],
)


TW = 4096
TGRID = (1000000 + TW - 1) // TW


def _tr_body(in_ref, out_ref):
    x = in_ref[...]
    r = lax.broadcasted_iota(jnp.int32, (D, D), 0)
    c = lax.broadcasted_iota(jnp.int32, (D, D), 1)
    eye = jnp.where(r == c, 1.0, 0.0)
    ats = lax.dot_general(x, eye, (((0,), (0,)), ((), ())),
                          preferred_element_type=jnp.float32)
    t3 = ats.reshape(TW // 2, 2, D)
    out_ref[:, 0:D] = t3[:, 0, :]
    out_ref[:, D:D2] = t3[:, 1, :]


def _pair_table(tab_t):
    return pl.pallas_call(
        _tr_body,
        grid=(TGRID,),
        in_specs=[pl.BlockSpec((D, TW), lambda i: (0, i))],
        out_specs=pl.BlockSpec((TW // 2, D2), lambda i: (i, 0)),
        out_shape=jax.ShapeDtypeStruct((500000, D2), jnp.float32),
    )(tab_t)


def _mlp_body(mix_ref, w1_ref, b1_ref, w2_ref, b2_ref, out_ref):
    h = jnp.dot(mix_ref[...], w1_ref[...], preferred_element_type=jnp.float32)
    h = jnp.maximum(h + b1_ref[...], 0.0)
    z = jnp.dot(h, w2_ref[...], preferred_element_type=jnp.float32)
    out_ref[...] = jax.nn.sigmoid(z + b2_ref[...])


def _mlp(mix, W1, b1, W2, b2):
    return pl.pallas_call(
        _mlp_body,
        grid=(GRID,),
        in_specs=[
            pl.BlockSpec((BLK, D), lambda i: (i, 0)),
            pl.BlockSpec((D, H), lambda i: (0, 0)),
            pl.BlockSpec((1, H), lambda i: (0, 0)),
            pl.BlockSpec((H, 1), lambda i: (0, 0)),
            pl.BlockSpec((1, 1), lambda i: (0, 0)),
        ],
        out_specs=pl.BlockSpec((BLK, 1), lambda i: (i, 0)),
        out_shape=jax.ShapeDtypeStruct((B, 1), jnp.float32),
    )(mix, W1, b1.reshape(1, H), W2, b2.reshape(1, 1))


def kernel(user, item, user_table, item_table, W1, b1, W2, b2):
    user = user.astype(jnp.int32)
    item = item.astype(jnp.int32)
    ut2 = _pair_table(user_table.T)
    it2 = _pair_table(item_table.T)
    mix = _mix_call(user, item, ut2, it2)
    out = _mlp(mix, W1, b1, W2, b2)
    return out.reshape(-1)


# speedup vs baseline: 1.3277x; 1.3277x over previous
import jax
import jax.numpy as jnp
from jax import lax
from jax.experimental import pallas as pl
from jax.experimental.pallas import tpu as pltpu
from jax.experimental.pallas import tpu_sc as plsc

NC = 2
NS = 16
L = 16
NW = NC * NS

B = 16384
D = 64
H = 256
BPW = B // NW

CH = 256
NCHUNK = BPW // CH

BLK = 2048
GRID = B // BLK


def _mix_body(uidx_hbm, iidx_hbm, utab_hbm, itab_hbm, out_hbm,
              uidx_v, iidx_v, urows_v, irows_v, usem, isem):
    wid = lax.axis_index("s") * NC + lax.axis_index("c")
    base = wid * BPW
    pltpu.sync_copy(uidx_hbm.at[pl.ds(base, BPW)], uidx_v)
    pltpu.sync_copy(iidx_hbm.at[pl.ds(base, BPW)], iidx_v)

    def chunk(c, carry0):
        cbase = c * CH

        def issue16(k, carry):
            uvec = uidx_v[pl.ds(cbase + k * L, L)]
            ivec = iidx_v[pl.ds(cbase + k * L, L)]
            for j in range(L):
                u = uvec[j]
                pltpu.async_copy(utab_hbm.at[pl.ds(u, 1)],
                                 urows_v.at[pl.ds(k * L + j, 1)], usem)
                it = ivec[j]
                pltpu.async_copy(itab_hbm.at[pl.ds(it, 1)],
                                 irows_v.at[pl.ds(k * L + j, 1)], isem)
            return carry

        lax.fori_loop(0, CH // L, issue16, 0)
        pltpu.make_async_copy(utab_hbm.at[pl.ds(0, CH)], urows_v, usem).wait()
        pltpu.make_async_copy(itab_hbm.at[pl.ds(0, CH)], irows_v, isem).wait()

        def row(i, carry):
            for j in range(D // L):
                sl = (i, pl.ds(j * L, L))
                urows_v[sl] = urows_v[sl] * irows_v[sl]
            pltpu.async_copy(urows_v.at[pl.ds(i, 1)],
                             out_hbm.at[pl.ds(base + cbase + i, 1)], isem)
            return carry

        lax.fori_loop(0, CH, row, 0)
        pltpu.make_async_copy(out_hbm.at[pl.ds(0, CH)], irows_v, isem).wait()
        return carry0

    lax.fori_loop(0, NCHUNK, chunk, 0)


_mix_call = pl.kernel(
    _mix_body,
    mesh=plsc.VectorSubcoreMesh(core_axis_name="c", subcore_axis_name="s"),
    out_type=jax.ShapeDtypeStruct((B, D), jnp.float32),
    scratch_types=[
        pltpu.VMEM((BPW,), jnp.int32),
        pltpu.VMEM((BPW,), jnp.int32),
        pltpu.VMEM((CH, D), jnp.float32),
        pltpu.VMEM((CH, D), jnp.float32),
        pltpu.SemaphoreType.DMA,
        pltpu.SemaphoreType.DMA,
    ],
)


def _mlp_body(mix_ref, w1_ref, b1_ref, w2_ref, b2_ref, out_ref):
    h = jnp.dot(mix_ref[...], w1_ref[...], preferred_element_type=jnp.float32)
    h = jnp.maximum(h + b1_ref[...], 0.0)
    z = jnp.dot(h, w2_ref[...], preferred_element_type=jnp.float32)
    out_ref[...] = jax.nn.sigmoid(z + b2_ref[...])


def _mlp(mix, W1, b1, W2, b2):
    return pl.pallas_call(
        _mlp_body,
        grid=(GRID,),
        in_specs=[
            pl.BlockSpec((BLK, D), lambda i: (i, 0)),
            pl.BlockSpec((D, H), lambda i: (0, 0)),
            pl.BlockSpec((1, H), lambda i: (0, 0)),
            pl.BlockSpec((H, 1), lambda i: (0, 0)),
            pl.BlockSpec((1, 1), lambda i: (0, 0)),
        ],
        out_specs=pl.BlockSpec((BLK, 1), lambda i: (i, 0)),
        out_shape=jax.ShapeDtypeStruct((B, 1), jnp.float32),
    )(mix, W1, b1.reshape(1, H), W2, b2.reshape(1, 1))


def kernel(user, item, user_table, item_table, W1, b1, W2, b2):
    user = user.astype(jnp.int32)
    item = item.astype(jnp.int32)
    mix = _mix_call(user, item, user_table, item_table)
    out = _mlp(mix, W1, b1, W2, b2)
    return out.reshape(-1)
